# baseline (speedup 1.0000x reference)
import functools

import jax
import jax.numpy as jnp
from jax import lax
from jax.experimental import pallas as pl
from jax.experimental.pallas import tpu as pltpu
from jax.experimental.pallas import tpu_sc as plsc

_NC = 2
_NS = 16
_NW = _NC * _NS
_L = 16
_K = 8
_P = 2
_SC_ROWS = 13312


def _body(m, batch, n, x_hbm, perm_hbm, y_hbm, ld_hbm,
          perm_v, in0, in1, out0, out1, zv,
          si0, si1, so0, so1):
    rows_per_tile = m // _NW
    nch = rows_per_tile // _K
    halfn = n // _P
    nj = halfn // _L
    ld_rows = batch // _NW
    cid = lax.axis_index("c")
    sid = lax.axis_index("s")
    wid = sid * _NC + cid
    row0 = wid * rows_per_tile
    ldrow0 = wid * ld_rows

    pltpu.sync_copy(perm_hbm, perm_v)

    zvec = jnp.zeros((_L,), jnp.float32)

    def _zero(i, carry):
        zv[pl.ds(i * _L, _L)] = zvec
        return carry

    lax.fori_loop(0, ld_rows // _L, _zero, 0)
    pltpu.sync_copy(zv, ld_hbm.at[pl.ds(ldrow0, ld_rows)])

    ins = (in0, in1)
    outs = (out0, out1)
    isems = (si0, si1)
    osems = (so0, so1)

    def in_copy(c, b):
        return pltpu.make_async_copy(
            x_hbm.at[pl.ds(row0 + c * _K, _K)], ins[b], isems[b])

    def out_copy(c, p):
        return pltpu.make_async_copy(
            outs[p],
            y_hbm.at[pl.ds(row0 + c * _K, _K), pl.ds(p * halfn, halfn)],
            osems[p])

    in_copy(0, 0).start()

    rvec = [jnp.full((_L,), r, jnp.int32) for r in range(_K)]

    def chunk_pair(h, carry):
        for b in range(2):
            c = h * 2 + b

            @pl.when(c + 1 < nch)
            def _start_next():
                in_copy(c + 1, 1 - b).start()

            in_copy(c, b).wait()
            ib = ins[b]

            for p in range(_P):
                @pl.when(c >= 1)
                def _free_out():
                    out_copy(c - 1, p).wait()

                ob = outs[p]
                j0 = p * nj

                @plsc.parallel_loop(0, nj, unroll=8)
                def _jbody(jj):
                    idx = perm_v[pl.ds((j0 + jj) * _L, _L)]
                    for r in range(_K):
                        ob[r, pl.ds(jj * _L, _L)] = (
                            plsc.load_gather(ib, [rvec[r], idx]))
                out_copy(c, p).start()
        return carry

    lax.fori_loop(0, nch // 2, chunk_pair, 0)
    out_copy(nch - 1, 0).wait()
    out_copy(nch - 1, 1).wait()


def kernel(x, perm):
    batch, n = x.shape
    perm = perm.astype(jnp.int32)
    mesh = plsc.VectorSubcoreMesh(core_axis_name="c", subcore_axis_name="s")
    call = pl.kernel(
        functools.partial(_body, _SC_ROWS, batch, n),
        out_type=(
            jax.ShapeDtypeStruct((batch, n), x.dtype),
            jax.ShapeDtypeStruct((batch,), x.dtype),
        ),
        mesh=mesh,
        compiler_params=pltpu.CompilerParams(needs_layout_passes=False),
        scratch_types=[
            pltpu.VMEM((n,), jnp.int32),
            pltpu.VMEM((_K, n), jnp.float32),
            pltpu.VMEM((_K, n), jnp.float32),
            pltpu.VMEM((_K, n // _P), jnp.float32),
            pltpu.VMEM((_K, n // _P), jnp.float32),
            pltpu.VMEM((batch // _NW,), jnp.float32),
            pltpu.SemaphoreType.DMA,
            pltpu.SemaphoreType.DMA,
            pltpu.SemaphoreType.DMA,
            pltpu.SemaphoreType.DMA,
        ],
    )
    y_sc, log_det = call(x, perm)
    y_tc = jnp.take(x[_SC_ROWS:], perm, axis=1)
    y = lax.dynamic_update_slice(y_sc, y_tc, (_SC_ROWS, 0))
    return y, log_det


# speedup vs baseline: 1.6932x; 1.6932x over previous
import functools

import jax
import jax.numpy as jnp
from jax import lax
from jax.experimental import pallas as pl
from jax.experimental.pallas import tpu as pltpu
from jax.experimental.pallas import tpu_sc as plsc

_NC = 2
_NS = 16
_NW = _NC * _NS
_L = 16
_K = 8
_P = 2


def _body(batch, n, x_hbm, perm_hbm, y_hbm, ld_hbm,
          perm_v, in0, in1, out0, out1, zv,
          si0, si1, so0, so1):
    rows_per_tile = batch // _NW
    nch = rows_per_tile // _K
    halfn = n // _P
    nj = halfn // _L
    cid = lax.axis_index("c")
    sid = lax.axis_index("s")
    wid = sid * _NC + cid
    ldrow0 = wid * rows_per_tile

    pltpu.sync_copy(perm_hbm, perm_v)

    zvec = jnp.zeros((_L,), jnp.float32)

    def _zero(i, carry):
        zv[pl.ds(i * _L, _L)] = zvec
        return carry

    lax.fori_loop(0, rows_per_tile // _L, _zero, 0)
    pltpu.sync_copy(zv, ld_hbm.at[pl.ds(ldrow0, rows_per_tile)])

    ins = (in0, in1)
    outs = (out0, out1)
    isems = (si0, si1)
    osems = (so0, so1)

    def crow(c):
        return (c * _NW + wid) * _K

    def in_copy(c, b):
        return pltpu.make_async_copy(
            x_hbm.at[pl.ds(crow(c), _K)], ins[b], isems[b])

    def out_copy(c, p):
        return pltpu.make_async_copy(
            outs[p],
            y_hbm.at[pl.ds(crow(c), _K), pl.ds(p * halfn, halfn)],
            osems[p])

    in_copy(0, 0).start()

    rvec = [jnp.full((_L,), r, jnp.int32) for r in range(_K)]

    def chunk_pair(h, carry):
        for b in range(2):
            c = h * 2 + b

            @pl.when(c + 1 < nch)
            def _start_next():
                in_copy(c + 1, 1 - b).start()

            in_copy(c, b).wait()
            ib = ins[b]

            for p in range(_P):
                @pl.when(c >= 1)
                def _free_out():
                    out_copy(c - 1, p).wait()

                ob = outs[p]
                j0 = p * nj

                @plsc.parallel_loop(0, nj, unroll=8)
                def _jbody(jj):
                    idx = perm_v[pl.ds((j0 + jj) * _L, _L)]
                    for r in range(_K):
                        ob[r, pl.ds(jj * _L, _L)] = (
                            plsc.load_gather(ib, [rvec[r], idx]))
                out_copy(c, p).start()
        return carry

    lax.fori_loop(0, nch // 2, chunk_pair, 0)
    out_copy(nch - 1, 0).wait()
    out_copy(nch - 1, 1).wait()


def kernel(x, perm):
    batch, n = x.shape
    perm = perm.astype(jnp.int32)
    mesh = plsc.VectorSubcoreMesh(core_axis_name="c", subcore_axis_name="s")
    call = pl.kernel(
        functools.partial(_body, batch, n),
        out_type=(
            jax.ShapeDtypeStruct((batch, n), x.dtype),
            jax.ShapeDtypeStruct((batch,), x.dtype),
        ),
        mesh=mesh,
        compiler_params=pltpu.CompilerParams(needs_layout_passes=False),
        scratch_types=[
            pltpu.VMEM((n,), jnp.int32),
            pltpu.VMEM((_K, n), jnp.float32),
            pltpu.VMEM((_K, n), jnp.float32),
            pltpu.VMEM((_K, n // _P), jnp.float32),
            pltpu.VMEM((_K, n // _P), jnp.float32),
            pltpu.VMEM((batch // _NW,), jnp.float32),
            pltpu.SemaphoreType.DMA,
            pltpu.SemaphoreType.DMA,
            pltpu.SemaphoreType.DMA,
            pltpu.SemaphoreType.DMA,
        ],
    )
    y, log_det = call(x, perm)
    return y, log_det


# speedup vs baseline: 1.7024x; 1.0054x over previous
import functools

import jax
import jax.numpy as jnp
from jax import lax
from jax.experimental import pallas as pl
from jax.experimental.pallas import tpu as pltpu
from jax.experimental.pallas import tpu_sc as plsc

_NC = 2
_NS = 16
_NW = _NC * _NS
_L = 16
_K = 8
_P = 2


def _body(batch, n, x_hbm, perm_hbm, y_hbm, ld_hbm,
          perm_v, in0, in1, out0, out1, zv,
          si0, si1, so0, so1):
    rows_per_tile = batch // _NW
    nch = rows_per_tile // _K
    halfn = n // _P
    nj = halfn // _L
    cid = lax.axis_index("c")
    sid = lax.axis_index("s")
    wid = sid * _NC + cid
    row0 = wid * rows_per_tile

    pltpu.sync_copy(perm_hbm, perm_v)

    zvec = jnp.zeros((_L,), jnp.float32)

    def _zero(i, carry):
        zv[pl.ds(i * _L, _L)] = zvec
        return carry

    lax.fori_loop(0, rows_per_tile // _L, _zero, 0)
    pltpu.sync_copy(zv, ld_hbm.at[pl.ds(row0, rows_per_tile)])

    ins = (in0, in1)
    outs = (out0, out1)
    isems = (si0, si1)
    osems = (so0, so1)

    def crow(c):
        return row0 + c * _K

    def in_copy(c, b):
        return pltpu.make_async_copy(
            x_hbm.at[pl.ds(crow(c), _K)], ins[b], isems[b])

    def out_copy(c, p):
        return pltpu.make_async_copy(
            outs[p],
            y_hbm.at[pl.ds(crow(c), _K), pl.ds(p * halfn, halfn)],
            osems[p])

    in_copy(0, 0).start()

    rvec = [jnp.full((_L,), r, jnp.int32) for r in range(_K)]

    def chunk_pair(h, carry):
        for b in range(2):
            c = h * 2 + b

            @pl.when(c + 1 < nch)
            def _start_next():
                in_copy(c + 1, 1 - b).start()

            in_copy(c, b).wait()
            ib = ins[b]

            for p in range(_P):
                @pl.when(c >= 1)
                def _free_out():
                    out_copy(c - 1, p).wait()

                ob = outs[p]
                j0 = p * nj

                @plsc.parallel_loop(0, nj, unroll=8)
                def _jbody(jj):
                    idx = perm_v[pl.ds((j0 + jj) * _L, _L)]
                    for r in range(_K):
                        ob[r, pl.ds(jj * _L, _L)] = (
                            plsc.load_gather(ib, [rvec[r], idx]))
                out_copy(c, p).start()
        return carry

    lax.fori_loop(0, nch // 2, chunk_pair, 0)
    out_copy(nch - 1, 0).wait()
    out_copy(nch - 1, 1).wait()


def kernel(x, perm):
    batch, n = x.shape
    perm = perm.astype(jnp.int32)
    mesh = plsc.VectorSubcoreMesh(core_axis_name="c", subcore_axis_name="s")
    call = pl.kernel(
        functools.partial(_body, batch, n),
        out_type=(
            jax.ShapeDtypeStruct((batch, n), x.dtype),
            jax.ShapeDtypeStruct((batch,), x.dtype),
        ),
        mesh=mesh,
        compiler_params=pltpu.CompilerParams(needs_layout_passes=False),
        scratch_types=[
            pltpu.VMEM((n,), jnp.int32),
            pltpu.VMEM((_K, n), jnp.float32),
            pltpu.VMEM((_K, n), jnp.float32),
            pltpu.VMEM((_K, n // _P), jnp.float32),
            pltpu.VMEM((_K, n // _P), jnp.float32),
            pltpu.VMEM((batch // _NW,), jnp.float32),
            pltpu.SemaphoreType.DMA,
            pltpu.SemaphoreType.DMA,
            pltpu.SemaphoreType.DMA,
            pltpu.SemaphoreType.DMA,
        ],
    )
    y, log_det = call(x, perm)
    return y, log_det
